# baseline (speedup 1.0000x reference)
import jax
import jax.numpy as jnp
from jax import lax
from jax.experimental import pallas as pl
from jax.experimental.pallas import tpu as pltpu
from jax.experimental.pallas import tpu_sc as plsc

B, S, D, E, TOPK = 4, 8192, 768, 64, 2
BS = B * S
BM = 2048
NW = 32
TPW = BS // NW
L = 16
CHUNK = 256


def _scores_kernel(xr_ref, xi_ref, wa_ref, wp_ref, b_ref, scores_ref):
    xr = xr_ref[...]
    xi = xi_ref[...]
    amp = jnp.sqrt(xr * xr + xi * xi)
    phase = jnp.arctan2(xi, xr)
    scores = (
        jnp.dot(amp, wa_ref[...], preferred_element_type=jnp.float32)
        + jnp.dot(phase, wp_ref[...], preferred_element_type=jnp.float32)
        + b_ref[...]
    )
    scores_ref[...] = scores.T


def _sc_topk_kernel(scores_hbm, p0_hbm, p1_hbm, i1_hbm, i2_hbm,
                    scores_v, p0_v, p1_v, i1_v, i2_v):
    wid = lax.axis_index("s") * 2 + lax.axis_index("c")
    base = wid * TPW

    def chunk_body(c, _):
        pltpu.sync_copy(
            scores_hbm.at[:, pl.ds(base + c * CHUNK, CHUNK)], scores_v)

        def group_body(g, _):
            sl = pl.ds(g * L, L)
            m1 = scores_v[0, sl]
            i1 = jnp.zeros((L,), jnp.int32)
            m2 = jnp.full((L,), -jnp.inf, jnp.float32)
            i2 = jnp.zeros((L,), jnp.int32)
            for e in range(1, E):
                ev = jnp.full((L,), e, jnp.int32)
                s = scores_v[e, sl]
                gt1 = s > m1
                gt2 = s > m2
                i2 = jnp.where(gt1, i1, jnp.where(gt2, ev, i2))
                m2 = jnp.where(gt1, m1, jnp.where(gt2, s, m2))
                i1 = jnp.where(gt1, ev, i1)
                m1 = jnp.where(gt1, s, m1)
            p0 = 1.0 / (1.0 + jnp.exp(m2 - m1))
            osl = pl.ds(c * CHUNK + g * L, L)
            p0_v[osl] = p0
            p1_v[osl] = 1.0 - p0
            i1_v[osl] = i1
            i2_v[osl] = i2
            return 0

        lax.fori_loop(0, CHUNK // L, group_body, 0)
        return 0

    lax.fori_loop(0, TPW // CHUNK, chunk_body, 0)
    pltpu.sync_copy(p0_v, p0_hbm.at[pl.ds(base, TPW)])
    pltpu.sync_copy(p1_v, p1_hbm.at[pl.ds(base, TPW)])
    pltpu.sync_copy(i1_v, i1_hbm.at[pl.ds(base, TPW)])
    pltpu.sync_copy(i2_v, i2_hbm.at[pl.ds(base, TPW)])


@jax.jit
def kernel(x_real, x_imag, W, b):
    xr = x_real.reshape(BS, D)
    xi = x_imag.reshape(BS, D)
    wa = W[:D]
    wp = W[D:]
    b2 = b.reshape(1, E)

    scores_t = pl.pallas_call(
        _scores_kernel,
        grid=(BS // BM,),
        in_specs=[
            pl.BlockSpec((BM, D), lambda i: (i, 0)),
            pl.BlockSpec((BM, D), lambda i: (i, 0)),
            pl.BlockSpec((D, E), lambda i: (0, 0)),
            pl.BlockSpec((D, E), lambda i: (0, 0)),
            pl.BlockSpec((1, E), lambda i: (0, 0)),
        ],
        out_specs=pl.BlockSpec((E, BM), lambda i: (0, i)),
        out_shape=jax.ShapeDtypeStruct((E, BS), jnp.float32),
        compiler_params=pltpu.CompilerParams(
            dimension_semantics=("arbitrary",),
        ),
    )(xr, xi, wa, wp, b2)

    mesh = plsc.VectorSubcoreMesh(core_axis_name="c", subcore_axis_name="s")
    p0, p1, i1, i2 = pl.kernel(
        _sc_topk_kernel,
        out_type=[
            jax.ShapeDtypeStruct((BS,), jnp.float32),
            jax.ShapeDtypeStruct((BS,), jnp.float32),
            jax.ShapeDtypeStruct((BS,), jnp.int32),
            jax.ShapeDtypeStruct((BS,), jnp.int32),
        ],
        mesh=mesh,
        scratch_types=[
            pltpu.VMEM((E, CHUNK), jnp.float32),
            pltpu.VMEM((TPW,), jnp.float32),
            pltpu.VMEM((TPW,), jnp.float32),
            pltpu.VMEM((TPW,), jnp.int32),
            pltpu.VMEM((TPW,), jnp.int32),
        ],
    )(scores_t)

    probs = jnp.stack([p0, p1], axis=-1).reshape(B, S, TOPK)
    idx = jnp.stack([i1, i2], axis=-1).reshape(B, S, TOPK)
    return probs, idx


# speedup vs baseline: 1.0065x; 1.0065x over previous
import jax
import jax.numpy as jnp
from jax import lax
from jax.experimental import pallas as pl
from jax.experimental.pallas import tpu as pltpu
from jax.experimental.pallas import tpu_sc as plsc

B, S, D, E, TOPK = 4, 8192, 768, 64, 2
BS = B * S
BM = 2048
NW = 32
TPW = BS // NW
L = 16
CHUNK = 256


def _scores_kernel(xr_ref, xi_ref, w_ref, b_ref, scores_ref):
    xr = xr_ref[...]
    xi = xi_ref[...]
    amp = jnp.sqrt(xr * xr + xi * xi)
    phase = jnp.arctan2(xi, xr)
    feat = jnp.concatenate([amp, phase], axis=1)
    scores = (
        jnp.dot(feat, w_ref[...], preferred_element_type=jnp.float32)
        + b_ref[...]
    )
    scores_ref[...] = scores.T


def _sc_topk_kernel(scores_hbm, p0_hbm, p1_hbm, i1_hbm, i2_hbm,
                    scores_v, p0_v, p1_v, i1_v, i2_v):
    wid = lax.axis_index("s") * 2 + lax.axis_index("c")
    base = wid * TPW

    def chunk_body(c, _):
        pltpu.sync_copy(
            scores_hbm.at[:, pl.ds(base + c * CHUNK, CHUNK)], scores_v)

        def group_body(g, _):
            sl = pl.ds(g * L, L)
            m1 = scores_v[0, sl]
            i1 = jnp.zeros((L,), jnp.int32)
            m2 = jnp.full((L,), -jnp.inf, jnp.float32)
            i2 = jnp.zeros((L,), jnp.int32)
            for e in range(1, E):
                ev = jnp.full((L,), e, jnp.int32)
                s = scores_v[e, sl]
                gt1 = s > m1
                gt2 = s > m2
                i2 = jnp.where(gt1, i1, jnp.where(gt2, ev, i2))
                m2 = jnp.where(gt1, m1, jnp.where(gt2, s, m2))
                i1 = jnp.where(gt1, ev, i1)
                m1 = jnp.where(gt1, s, m1)
            p0 = 1.0 / (1.0 + jnp.exp(m2 - m1))
            osl = pl.ds(c * CHUNK + g * L, L)
            p0_v[osl] = p0
            p1_v[osl] = 1.0 - p0
            i1_v[osl] = i1
            i2_v[osl] = i2
            return 0

        lax.fori_loop(0, CHUNK // L, group_body, 0)
        return 0

    lax.fori_loop(0, TPW // CHUNK, chunk_body, 0)
    pltpu.sync_copy(p0_v, p0_hbm.at[pl.ds(base, TPW)])
    pltpu.sync_copy(p1_v, p1_hbm.at[pl.ds(base, TPW)])
    pltpu.sync_copy(i1_v, i1_hbm.at[pl.ds(base, TPW)])
    pltpu.sync_copy(i2_v, i2_hbm.at[pl.ds(base, TPW)])


@jax.jit
def kernel(x_real, x_imag, W, b):
    xr = x_real.reshape(BS, D)
    xi = x_imag.reshape(BS, D)
    b2 = b.reshape(1, E)

    scores_t = pl.pallas_call(
        _scores_kernel,
        grid=(BS // BM,),
        in_specs=[
            pl.BlockSpec((BM, D), lambda i: (i, 0)),
            pl.BlockSpec((BM, D), lambda i: (i, 0)),
            pl.BlockSpec((2 * D, E), lambda i: (0, 0)),
            pl.BlockSpec((1, E), lambda i: (0, 0)),
        ],
        out_specs=pl.BlockSpec((E, BM), lambda i: (0, i)),
        out_shape=jax.ShapeDtypeStruct((E, BS), jnp.float32),
        compiler_params=pltpu.CompilerParams(
            dimension_semantics=("arbitrary",),
        ),
    )(xr, xi, W, b2)

    mesh = plsc.VectorSubcoreMesh(core_axis_name="c", subcore_axis_name="s")
    p0, p1, i1, i2 = pl.kernel(
        _sc_topk_kernel,
        out_type=[
            jax.ShapeDtypeStruct((BS,), jnp.float32),
            jax.ShapeDtypeStruct((BS,), jnp.float32),
            jax.ShapeDtypeStruct((BS,), jnp.int32),
            jax.ShapeDtypeStruct((BS,), jnp.int32),
        ],
        mesh=mesh,
        scratch_types=[
            pltpu.VMEM((E, CHUNK), jnp.float32),
            pltpu.VMEM((TPW,), jnp.float32),
            pltpu.VMEM((TPW,), jnp.float32),
            pltpu.VMEM((TPW,), jnp.int32),
            pltpu.VMEM((TPW,), jnp.int32),
        ],
    )(scores_t)

    probs = jnp.stack([p0, p1], axis=-1).reshape(B, S, TOPK)
    idx = jnp.stack([i1, i2], axis=-1).reshape(B, S, TOPK)
    return probs, idx
